# baseline (speedup 1.0000x reference)
import functools

import jax
import jax.numpy as jnp
from jax import lax
from jax.experimental import pallas as pl
from jax.experimental.pallas import tpu as pltpu
from jax.experimental.pallas import tpu_sc as plsc

_E = 8
_SCALE = 0.01
_BT = 128
_NH = 2
_NC = 2
_NSC = 16
_NW = _NC * _NSC


def _sc_mesh():
    return plsc.VectorSubcoreMesh(core_axis_name="c", subcore_axis_name="s",
                                  num_cores=_NC, num_subcores=_NSC)


_SC_PARAMS = pltpu.CompilerParams(needs_layout_passes=False)


def _route_body(T, lt_ref, topi_ref, dest_ref, off_ref,
                lt_v, be_v, dst_v, stage_v, ctr_v):
    cid = lax.axis_index("c")
    sid = lax.axis_index("s")
    lanes = jnp.arange(16, dtype=jnp.int32)
    ng = T // 16

    @pl.when(jnp.logical_and(cid == 0, sid == 0))
    def _():
        pltpu.sync_copy(lt_ref, lt_v)
        ctr_v[...] = jnp.zeros((16,), jnp.int32)

        def amax_body(g, c):
            sl = pl.ds(g * 16, 16)
            m = lt_v[0, sl]
            be = jnp.zeros((16,), jnp.int32)
            for e in range(1, _E):
                v = lt_v[e, sl]
                gt = v > m
                m = jnp.where(gt, v, m)
                be = jnp.where(gt, e, be)
            be_v[sl] = be
            rank, lastm = plsc.scan_count(be)
            base = plsc.load_gather(ctr_v, [be])
            dst_v[sl] = base + rank - 1
            plsc.store_scatter(ctr_v, [be], base + rank, mask=lastm)
            return c

        lax.fori_loop(0, ng, amax_body, 0)
        pltpu.sync_copy(be_v, topi_ref)

        cnt = ctr_v[...]
        inc = plsc.cumsum(cnt)
        exc = inc - cnt
        stage_v[...] = jnp.where(lanes < _E, exc, T)
        pltpu.sync_copy(stage_v, off_ref)

        def add_off_body(g, c):
            sl = pl.ds(g * 16, 16)
            beg = be_v[sl]
            o = plsc.load_gather(stage_v, [beg])
            dst_v[sl] = jnp.clip(dst_v[sl] + o, 0, T - 1)
            return c

        lax.fori_loop(0, ng, add_off_body, 0)
        pltpu.sync_copy(dst_v, dest_ref)


def _gather_rows_body(T, table_ref, idx_ref, out_ref, idx_v, rows_v, sem):
    wid = lax.axis_index("s") * _NC + lax.axis_index("c")
    bpw = T // _NW
    base = wid * bpw
    pltpu.sync_copy(idx_ref.at[pl.ds(base, bpw)], idx_v)
    pltpu.async_copy(table_ref.at[idx_v], rows_v, sem).wait()
    pltpu.sync_copy(rows_v, out_ref.at[pl.ds(base, bpw)])


def _sc_route(logitsT, T):
    tpw = T // _NSC
    fn = pl.kernel(
        functools.partial(_route_body, T),
        out_type=[jax.ShapeDtypeStruct((T,), jnp.int32),
                  jax.ShapeDtypeStruct((T,), jnp.int32),
                  jax.ShapeDtypeStruct((16,), jnp.int32)],
        mesh=_sc_mesh(),
        compiler_params=_SC_PARAMS,
        scratch_types=[pltpu.VMEM((_E, T), jnp.float32),
                       pltpu.VMEM((T,), jnp.int32),
                       pltpu.VMEM((T,), jnp.int32),
                       pltpu.VMEM((16,), jnp.int32),
                       pltpu.VMEM((16,), jnp.int32)],
    )
    return fn(logitsT)


def _sc_gather_rows(table, idx, T, D):
    bpw = T // _NW
    fn = pl.kernel(
        functools.partial(_gather_rows_body, T),
        out_type=jax.ShapeDtypeStruct((T, D), jnp.float32),
        mesh=_sc_mesh(),
        compiler_params=_SC_PARAMS,
        scratch_types=[pltpu.VMEM((bpw,), jnp.int32),
                       pltpu.VMEM((bpw, D), jnp.float32),
                       pltpu.SemaphoreType.DMA],
    )
    return fn(table, idx)


def _ffn_body(off_ref, x_ref, w1_ref, b1_ref, w2_ref, b2_ref, wp_ref, bp_ref,
              out_ref):
    e = pl.program_id(0)
    h = pl.program_id(1)
    lo = off_ref[e]
    hi = off_ref[e + 1]
    astart = (lo // 8) * 8
    nch = pl.cdiv(hi - astart, _BT)
    T = x_ref.shape[0]

    def chunk(i, _):
        ustart = astart + i * _BT
        start = jnp.minimum(ustart, T - _BT)
        xb = x_ref[pl.ds(start, _BT), :]
        h1 = jnp.dot(xb, w1_ref[0], preferred_element_type=jnp.float32)
        h1 = h1 + b1_ref[0]
        h2 = jnp.dot(xb, w2_ref[0], preferred_element_type=jnp.float32)
        h2 = h2 + b2_ref[0]
        act = h1 * (h2 * jax.nn.sigmoid(h2))
        o = jnp.dot(act, wp_ref[0], preferred_element_type=jnp.float32)
        p = start + jax.lax.broadcasted_iota(jnp.int32, (_BT, 1), 0)
        mask = jnp.logical_and(p >= jnp.maximum(lo, ustart), p < hi)
        prev = out_ref[pl.ds(start, _BT), :]

        @pl.when(h == 0)
        def _():
            out_ref[pl.ds(start, _BT), :] = jnp.where(mask, o + bp_ref[0],
                                                      prev)

        @pl.when(h != 0)
        def _():
            out_ref[pl.ds(start, _BT), :] = prev + jnp.where(mask, o, 0.0)

        return 0

    jax.lax.fori_loop(0, nch, chunk, 0)


def _grouped_ffn(x_sorted, W1, b1, W2, b2, Wp, bp, off):
    T, D = x_sorted.shape
    H = W1.shape[-1]
    HC = H // _NH
    grid_spec = pltpu.PrefetchScalarGridSpec(
        num_scalar_prefetch=1,
        grid=(_E, _NH),
        in_specs=[
            pl.BlockSpec((T, D), lambda e, h, off: (0, 0)),
            pl.BlockSpec((1, D, HC), lambda e, h, off: (e, 0, h)),
            pl.BlockSpec((1, 1, HC), lambda e, h, off: (e, 0, h)),
            pl.BlockSpec((1, D, HC), lambda e, h, off: (e, 0, h)),
            pl.BlockSpec((1, 1, HC), lambda e, h, off: (e, 0, h)),
            pl.BlockSpec((1, HC, D), lambda e, h, off: (e, h, 0)),
            pl.BlockSpec((1, 1, D), lambda e, h, off: (e, 0, 0)),
        ],
        out_specs=pl.BlockSpec((T, D), lambda e, h, off: (0, 0)),
    )
    return pl.pallas_call(
        _ffn_body,
        grid_spec=grid_spec,
        out_shape=jax.ShapeDtypeStruct((T, D), jnp.float32),
    )(off, x_sorted, W1,
      b1.reshape(b1.shape[0], 1, b1.shape[1]), W2,
      b2.reshape(b2.shape[0], 1, b2.shape[1]), Wp,
      bp.reshape(bp.shape[0], 1, bp.shape[1]))


def kernel(x, gate_W, noise_weight, W1, b1, W2, b2, Wp, bp, noise):
    x_flat = x.reshape(-1, x.shape[-1])
    T, D = x_flat.shape
    E = gate_W.shape[-1]
    logits = x_flat @ gate_W
    logits_noisy = logits + noise * noise_weight[None, :]
    gw_mean = jax.nn.softmax(logits, axis=-1).mean(axis=0)
    lb_loss = jnp.mean((gw_mean - 1.0 / E) ** 2) * _SCALE

    topi_flat, dest, off16 = _sc_route(logits_noisy.T, T)
    off = jnp.clip(off16[:E + 1], 0, T)
    src = jnp.zeros((T,), jnp.int32).at[dest].set(
        jnp.arange(T, dtype=jnp.int32), mode='drop')
    x_sorted = _sc_gather_rows(x_flat, src, T, D)
    out_sorted = _grouped_ffn(x_sorted, W1, b1, W2, b2, Wp, bp, off)
    final_flat = _sc_gather_rows(out_sorted, dest, T, D)
    final = final_flat.reshape(x.shape)
    return final, topi_flat[:, None], lb_loss


# speedup vs baseline: 1.0347x; 1.0347x over previous
import functools

import jax
import jax.numpy as jnp
from jax import lax
from jax.experimental import pallas as pl
from jax.experimental.pallas import tpu as pltpu
from jax.experimental.pallas import tpu_sc as plsc

_E = 8
_SCALE = 0.01
_BT = 128
_NH = 2
_NC = 2
_NSC = 16
_NW = _NC * _NSC


def _sc_mesh():
    return plsc.VectorSubcoreMesh(core_axis_name="c", subcore_axis_name="s",
                                  num_cores=_NC, num_subcores=_NSC)


_SC_PARAMS = pltpu.CompilerParams(needs_layout_passes=False)


def _route_body(T, lt_ref, topi_ref, dest_ref, src_ref, off_ref,
                lt_v, be_v, dst_v, src_v, stage_v, ctr_v):
    cid = lax.axis_index("c")
    sid = lax.axis_index("s")
    lanes = jnp.arange(16, dtype=jnp.int32)
    ng = T // 16

    @pl.when(jnp.logical_and(cid == 0, sid == 0))
    def _():
        pltpu.sync_copy(lt_ref, lt_v)
        ctr_v[...] = jnp.zeros((16,), jnp.int32)

        def amax_body(g, c):
            sl = pl.ds(g * 16, 16)
            m = lt_v[0, sl]
            be = jnp.zeros((16,), jnp.int32)
            for e in range(1, _E):
                v = lt_v[e, sl]
                gt = v > m
                m = jnp.where(gt, v, m)
                be = jnp.where(gt, e, be)
            be_v[sl] = be
            rank, lastm = plsc.scan_count(be)
            base = plsc.load_gather(ctr_v, [be])
            dst_v[sl] = base + rank - 1
            plsc.store_scatter(ctr_v, [be], base + rank, mask=lastm)
            return c

        lax.fori_loop(0, ng, amax_body, 0)
        pltpu.sync_copy(be_v, topi_ref)

        cnt = ctr_v[...]
        inc = plsc.cumsum(cnt)
        exc = inc - cnt
        stage_v[...] = jnp.where(lanes < _E, exc, T)
        pltpu.sync_copy(stage_v, off_ref)

        def add_off_body(g, c):
            sl = pl.ds(g * 16, 16)
            beg = be_v[sl]
            o = plsc.load_gather(stage_v, [beg])
            d = jnp.clip(dst_v[sl] + o, 0, T - 1)
            dst_v[sl] = d
            plsc.store_scatter(src_v, [d],
                               g * 16 + jnp.arange(16, dtype=jnp.int32))
            return c

        lax.fori_loop(0, ng, add_off_body, 0)
        pltpu.sync_copy(dst_v, dest_ref)
        pltpu.sync_copy(src_v, src_ref)


def _gather_rows_body(T, table_ref, idx_ref, out_ref, idx_v, rows_v, sem):
    wid = lax.axis_index("s") * _NC + lax.axis_index("c")
    bpw = T // _NW
    base = wid * bpw
    pltpu.sync_copy(idx_ref.at[pl.ds(base, bpw)], idx_v)
    pltpu.async_copy(table_ref.at[idx_v], rows_v, sem).wait()
    pltpu.sync_copy(rows_v, out_ref.at[pl.ds(base, bpw)])


def _sc_route(logitsT, T):
    tpw = T // _NSC
    fn = pl.kernel(
        functools.partial(_route_body, T),
        out_type=[jax.ShapeDtypeStruct((T,), jnp.int32),
                  jax.ShapeDtypeStruct((T,), jnp.int32),
                  jax.ShapeDtypeStruct((T,), jnp.int32),
                  jax.ShapeDtypeStruct((16,), jnp.int32)],
        mesh=_sc_mesh(),
        compiler_params=_SC_PARAMS,
        scratch_types=[pltpu.VMEM((_E, T), jnp.float32),
                       pltpu.VMEM((T,), jnp.int32),
                       pltpu.VMEM((T,), jnp.int32),
                       pltpu.VMEM((T,), jnp.int32),
                       pltpu.VMEM((16,), jnp.int32),
                       pltpu.VMEM((16,), jnp.int32)],
    )
    return fn(logitsT)


def _sc_gather_rows(table, idx, T, D):
    bpw = T // _NW
    fn = pl.kernel(
        functools.partial(_gather_rows_body, T),
        out_type=jax.ShapeDtypeStruct((T, D), jnp.float32),
        mesh=_sc_mesh(),
        compiler_params=_SC_PARAMS,
        scratch_types=[pltpu.VMEM((bpw,), jnp.int32),
                       pltpu.VMEM((bpw, D), jnp.float32),
                       pltpu.SemaphoreType.DMA],
    )
    return fn(table, idx)


def _ffn_body(off_ref, x_ref, w1_ref, b1_ref, w2_ref, b2_ref, wp_ref, bp_ref,
              out_ref):
    e = pl.program_id(0)
    h = pl.program_id(1)
    lo = off_ref[e]
    hi = off_ref[e + 1]
    astart = (lo // 8) * 8
    nch = pl.cdiv(hi - astart, _BT)
    T = x_ref.shape[0]

    def chunk(i, _):
        ustart = astart + i * _BT
        start = jnp.minimum(ustart, T - _BT)
        xb = x_ref[pl.ds(start, _BT), :]
        h1 = jnp.dot(xb, w1_ref[0], preferred_element_type=jnp.float32)
        h1 = h1 + b1_ref[0]
        h2 = jnp.dot(xb, w2_ref[0], preferred_element_type=jnp.float32)
        h2 = h2 + b2_ref[0]
        act = h1 * (h2 * jax.nn.sigmoid(h2))
        o = jnp.dot(act, wp_ref[0], preferred_element_type=jnp.float32)
        p = start + jax.lax.broadcasted_iota(jnp.int32, (_BT, 1), 0)
        mask = jnp.logical_and(p >= jnp.maximum(lo, ustart), p < hi)
        prev = out_ref[pl.ds(start, _BT), :]

        @pl.when(h == 0)
        def _():
            out_ref[pl.ds(start, _BT), :] = jnp.where(mask, o + bp_ref[0],
                                                      prev)

        @pl.when(h != 0)
        def _():
            out_ref[pl.ds(start, _BT), :] = prev + jnp.where(mask, o, 0.0)

        return 0

    jax.lax.fori_loop(0, nch, chunk, 0)


def _grouped_ffn(x_sorted, W1, b1, W2, b2, Wp, bp, off):
    T, D = x_sorted.shape
    H = W1.shape[-1]
    HC = H // _NH
    grid_spec = pltpu.PrefetchScalarGridSpec(
        num_scalar_prefetch=1,
        grid=(_E, _NH),
        in_specs=[
            pl.BlockSpec((T, D), lambda e, h, off: (0, 0)),
            pl.BlockSpec((1, D, HC), lambda e, h, off: (e, 0, h)),
            pl.BlockSpec((1, 1, HC), lambda e, h, off: (e, 0, h)),
            pl.BlockSpec((1, D, HC), lambda e, h, off: (e, 0, h)),
            pl.BlockSpec((1, 1, HC), lambda e, h, off: (e, 0, h)),
            pl.BlockSpec((1, HC, D), lambda e, h, off: (e, h, 0)),
            pl.BlockSpec((1, 1, D), lambda e, h, off: (e, 0, 0)),
        ],
        out_specs=pl.BlockSpec((T, D), lambda e, h, off: (0, 0)),
    )
    return pl.pallas_call(
        _ffn_body,
        grid_spec=grid_spec,
        out_shape=jax.ShapeDtypeStruct((T, D), jnp.float32),
    )(off, x_sorted, W1,
      b1.reshape(b1.shape[0], 1, b1.shape[1]), W2,
      b2.reshape(b2.shape[0], 1, b2.shape[1]), Wp,
      bp.reshape(bp.shape[0], 1, bp.shape[1]))


def kernel(x, gate_W, noise_weight, W1, b1, W2, b2, Wp, bp, noise):
    x_flat = x.reshape(-1, x.shape[-1])
    T, D = x_flat.shape
    E = gate_W.shape[-1]
    logits = x_flat @ gate_W
    logits_noisy = logits + noise * noise_weight[None, :]
    gw_mean = jax.nn.softmax(logits, axis=-1).mean(axis=0)
    lb_loss = jnp.mean((gw_mean - 1.0 / E) ** 2) * _SCALE

    topi_flat, dest, src, off16 = _sc_route(logits_noisy.T, T)
    off = jnp.clip(off16[:E + 1], 0, T)
    x_sorted = _sc_gather_rows(x_flat, src, T, D)
    out_sorted = _grouped_ffn(x_sorted, W1, b1, W2, b2, Wp, bp, off)
    final_flat = _sc_gather_rows(out_sorted, dest, T, D)
    final = final_flat.reshape(x.shape)
    return final, topi_flat[:, None], lb_loss


# speedup vs baseline: 1.0392x; 1.0043x over previous
import functools

import jax
import jax.numpy as jnp
from jax import lax
from jax.experimental import pallas as pl
from jax.experimental.pallas import tpu as pltpu
from jax.experimental.pallas import tpu_sc as plsc

_E = 8
_SCALE = 0.01
_BT = 128
_NH = 2
_NC = 2
_NSC = 16
_NW = _NC * _NSC


def _sc_mesh():
    return plsc.VectorSubcoreMesh(core_axis_name="c", subcore_axis_name="s",
                                  num_cores=_NC, num_subcores=_NSC)


_SC_PARAMS = pltpu.CompilerParams(needs_layout_passes=False)


def _route_body(T, lt_ref, topi_ref, dest_ref, src_ref, off_ref,
                lt_v, be_v, dst_v, src_v, stage_v, ctr_v):
    cid = lax.axis_index("c")
    sid = lax.axis_index("s")
    lanes = jnp.arange(16, dtype=jnp.int32)
    ng = T // 16

    @pl.when(jnp.logical_and(cid == 0, sid == 0))
    def _():
        pltpu.sync_copy(lt_ref, lt_v)
        ctr_v[...] = jnp.zeros((16,), jnp.int32)

        def amax_body(g, c):
            sl = pl.ds(g * 16, 16)
            m = lt_v[0, sl]
            be = jnp.zeros((16,), jnp.int32)
            for e in range(1, _E):
                v = lt_v[e, sl]
                gt = v > m
                m = jnp.where(gt, v, m)
                be = jnp.where(gt, e, be)
            be_v[sl] = be
            rank, lastm = plsc.scan_count(be)
            base = plsc.load_gather(ctr_v, [be])
            dst_v[sl] = base + rank - 1
            plsc.store_scatter(ctr_v, [be], base + rank, mask=lastm)
            return c

        lax.fori_loop(0, ng, amax_body, 0)
        pltpu.sync_copy(be_v, topi_ref)

        cnt = ctr_v[...]
        inc = plsc.cumsum(cnt)
        exc = inc - cnt
        stage_v[...] = jnp.where(lanes < _E, exc, T)
        pltpu.sync_copy(stage_v, off_ref)

        def add_off_body(g, c):
            sl = pl.ds(g * 16, 16)
            beg = be_v[sl]
            o = plsc.load_gather(stage_v, [beg])
            d = jnp.clip(dst_v[sl] + o, 0, T - 1)
            dst_v[sl] = d
            plsc.store_scatter(src_v, [d],
                               g * 16 + jnp.arange(16, dtype=jnp.int32))
            return c

        lax.fori_loop(0, ng, add_off_body, 0)
        pltpu.sync_copy(dst_v, dest_ref)
        pltpu.sync_copy(src_v, src_ref)


def _gather_rows_body(T, table_ref, idx_ref, out_ref, idx_v, rows_v, sem):
    wid = lax.axis_index("s") * _NC + lax.axis_index("c")
    bpw = T // _NW
    base = wid * bpw
    pltpu.sync_copy(idx_ref.at[pl.ds(base, bpw)], idx_v)
    pltpu.async_copy(table_ref.at[idx_v], rows_v, sem).wait()
    pltpu.sync_copy(rows_v, out_ref.at[pl.ds(base, bpw)])


def _sc_route(logitsT, T):
    fn = pl.kernel(
        functools.partial(_route_body, T),
        out_type=[jax.ShapeDtypeStruct((T,), jnp.int32),
                  jax.ShapeDtypeStruct((T,), jnp.int32),
                  jax.ShapeDtypeStruct((T,), jnp.int32),
                  jax.ShapeDtypeStruct((16,), jnp.int32)],
        mesh=_sc_mesh(),
        compiler_params=_SC_PARAMS,
        scratch_types=[pltpu.VMEM((_E, T), jnp.float32),
                       pltpu.VMEM((T,), jnp.int32),
                       pltpu.VMEM((T,), jnp.int32),
                       pltpu.VMEM((T,), jnp.int32),
                       pltpu.VMEM((16,), jnp.int32),
                       pltpu.VMEM((16,), jnp.int32)],
    )
    return fn(logitsT)


def _sc_gather_rows(table, idx, T, D):
    bpw = T // _NW
    fn = pl.kernel(
        functools.partial(_gather_rows_body, T),
        out_type=jax.ShapeDtypeStruct((T, D), jnp.float32),
        mesh=_sc_mesh(),
        compiler_params=_SC_PARAMS,
        scratch_types=[pltpu.VMEM((bpw,), jnp.int32),
                       pltpu.VMEM((bpw, D), jnp.float32),
                       pltpu.SemaphoreType.DMA],
    )
    return fn(table, idx)


def _ffn_body(off_ref, x_ref, w1_ref, b1_ref, w2_ref, b2_ref, wp_ref, bp_ref,
              out_ref):
    e = pl.program_id(0)
    h = pl.program_id(1)
    lo = off_ref[e]
    hi = off_ref[e + 1]
    astart = (lo // 8) * 8
    nch = pl.cdiv(hi - astart, _BT)
    T = x_ref.shape[0]

    def chunk(i, _):
        ustart = astart + i * _BT
        start = jnp.minimum(ustart, T - _BT)
        xb = x_ref[pl.ds(start, _BT), :]
        h1 = jnp.dot(xb, w1_ref[0], preferred_element_type=jnp.float32)
        h1 = h1 + b1_ref[0]
        h2 = jnp.dot(xb, w2_ref[0], preferred_element_type=jnp.float32)
        h2 = h2 + b2_ref[0]
        act = h1 * (h2 * jax.nn.sigmoid(h2))
        o = jnp.dot(act, wp_ref[0], preferred_element_type=jnp.float32)
        p = start + jax.lax.broadcasted_iota(jnp.int32, (_BT, 1), 0)
        mask = jnp.logical_and(p >= jnp.maximum(lo, ustart), p < hi)
        prev = out_ref[pl.ds(start, _BT), :]

        @pl.when(h == 0)
        def _():
            out_ref[pl.ds(start, _BT), :] = jnp.where(mask, o + bp_ref[0],
                                                      prev)

        @pl.when(h != 0)
        def _():
            out_ref[pl.ds(start, _BT), :] = prev + jnp.where(mask, o, 0.0)

        return 0

    jax.lax.fori_loop(0, nch, chunk, 0)


def _grouped_ffn(x_sorted, W1, b1, W2, b2, Wp, bp, off):
    T, D = x_sorted.shape
    H = W1.shape[-1]
    HC = H // _NH
    grid_spec = pltpu.PrefetchScalarGridSpec(
        num_scalar_prefetch=1,
        grid=(_E, _NH),
        in_specs=[
            pl.BlockSpec((T, D), lambda e, h, off: (0, 0)),
            pl.BlockSpec((1, D, HC), lambda e, h, off: (e, 0, h)),
            pl.BlockSpec((1, 1, HC), lambda e, h, off: (e, 0, h)),
            pl.BlockSpec((1, D, HC), lambda e, h, off: (e, 0, h)),
            pl.BlockSpec((1, 1, HC), lambda e, h, off: (e, 0, h)),
            pl.BlockSpec((1, HC, D), lambda e, h, off: (e, h, 0)),
            pl.BlockSpec((1, 1, D), lambda e, h, off: (e, 0, 0)),
        ],
        out_specs=pl.BlockSpec((T, D), lambda e, h, off: (0, 0)),
    )
    return pl.pallas_call(
        _ffn_body,
        grid_spec=grid_spec,
        out_shape=jax.ShapeDtypeStruct((T, D), jnp.float32),
    )(off, x_sorted, W1,
      b1.reshape(b1.shape[0], 1, b1.shape[1]), W2,
      b2.reshape(b2.shape[0], 1, b2.shape[1]), Wp,
      bp.reshape(bp.shape[0], 1, bp.shape[1]))


def kernel(x, gate_W, noise_weight, W1, b1, W2, b2, Wp, bp, noise):
    x_flat = x.reshape(-1, x.shape[-1])
    T, D = x_flat.shape
    E = gate_W.shape[-1]
    logits = x_flat @ gate_W
    logits_noisy = logits + noise * noise_weight[None, :]
    gw_mean = jax.nn.softmax(logits, axis=-1).mean(axis=0)
    lb_loss = jnp.mean((gw_mean - 1.0 / E) ** 2) * _SCALE

    topi_flat, dest, src, off16 = _sc_route(logits_noisy.T, T)
    off = jnp.clip(off16[:E + 1], 0, T)
    x_sorted = _sc_gather_rows(x_flat, src, T, D)
    out_sorted = _grouped_ffn(x_sorted, W1, b1, W2, b2, Wp, bp, off)
    final_flat = _sc_gather_rows(out_sorted, dest, T, D)
    final = final_flat.reshape(x.shape)
    return final, topi_flat[:, None], lb_loss
